# Initial kernel scaffold; baseline (speedup 1.0000x reference)
import jax
import jax.numpy as jnp
from jax.experimental import pallas as pl


def kernel(x, edge_index, edge_attr, u, batch, phi_e_W0, phi_e_b0, phi_e_W1, phi_e_b1, phi_v_W0, phi_v_b0, phi_v_W1, phi_v_b1, phi_u_W0, phi_u_b0, phi_u_W1, phi_u_b1):
    raise NotImplementedError("write your pallas kernel here")


# speedup vs baseline: 5.8955x; 5.8955x over previous
import functools

import jax
import jax.numpy as jnp
from jax import lax
from jax.experimental import pallas as pl
from jax.experimental.pallas import tpu as pltpu
from jax.experimental.pallas import tpu_sc as plsc

N = 10000
E = 320000
H = 128
G = 64

NC = 2
NS = 16
NW = NC * NS
EPW = E // NW
CH = 80
NCH = EPW // CH
NPAIR = (NCH - 1) // 2
TAIL = 2 * NPAIR
NP = 10240
TR = NP // NS
CW = 128

BN = 1000
NB = N // BN
BE = 2000
NEB = E // BE

_PREC = lax.Precision.HIGHEST


def _sp(x):
    return jnp.maximum(x, 0.0) + jnp.log1p(jnp.exp(-jnp.abs(x)))


def _dot(a, b):
    return jnp.dot(a, b, preferred_element_type=jnp.float32, precision=_PREC)


def _dot0(a, b):
    return lax.dot_general(a, b, (((0,), (0,)), ((), ())),
                           preferred_element_type=jnp.float32, precision=_PREC)


def _prep_body(x_ref, b_ref, u_ref, wr_ref, wc_ref, wu_ref, be0_ref,
               xrb_ref, xc_ref):
    ue = _dot(u_ref[...], wu_ref[...])
    b = b_ref[0]
    ohT = (lax.broadcasted_iota(jnp.int32, (G, BN), 0) == b).astype(jnp.float32)
    ubn = _dot0(ohT, ue)
    xb = x_ref[...]
    xrb_ref[...] = _dot(xb, wr_ref[...]) + ubn + be0_ref[...]
    xc_ref[...] = _dot(xb, wc_ref[...])


def _prep(x, batch3, u, wr, wc, wu, be0):
    full = lambda r, c: pl.BlockSpec((r, c), lambda i: (0, 0))
    return pl.pallas_call(
        _prep_body,
        grid=(NB,),
        in_specs=[
            pl.BlockSpec((BN, H), lambda i: (i, 0)),
            pl.BlockSpec((1, 1, BN), lambda i: (i, 0, 0)),
            full(G, H), full(H, H), full(H, H), full(H, H), full(1, H),
        ],
        out_specs=[pl.BlockSpec((BN, H), lambda i: (i, 0)),
                   pl.BlockSpec((BN, H), lambda i: (i, 0))],
        out_shape=[jax.ShapeDtypeStruct((N, H), jnp.float32)] * 2,
    )(x, batch3, u, wr, wc, wu, be0)


_MESH = plsc.VectorSubcoreMesh(core_axis_name="c", subcore_axis_name="s",
                               num_cores=NC, num_subcores=NS)


@functools.partial(
    pl.kernel,
    out_type=jax.ShapeDtypeStruct((E, H), jnp.float32),
    mesh=_MESH,
    scratch_types=[
        pltpu.VMEM((NCH, CH), jnp.int32),
        pltpu.VMEM((NCH, CH), jnp.int32),
        pltpu.VMEM((CH, H), jnp.float32),
        pltpu.VMEM((CH, H), jnp.float32),
        pltpu.SemaphoreType.DMA,
        pltpu.SemaphoreType.DMA,
    ],
)
def _sc_gather(xrb_hbm, xc_hbm, row_hbm, col_hbm, base_hbm,
               idx_r, idx_c, buf_a, buf_b, sem_a, sem_b):
    c = lax.axis_index("c")
    s = lax.axis_index("s")
    wid = s * NC + c
    ebase = wid * EPW
    pltpu.sync_copy(row_hbm.at[wid], idx_r)
    pltpu.sync_copy(col_hbm.at[wid], idx_c)

    def pair(i, carry):
        j0 = 2 * i
        j1 = j0 + 1
        a = pltpu.async_copy(xrb_hbm.at[idx_r.at[j0]], buf_a, sem_a)
        b = pltpu.async_copy(xrb_hbm.at[idx_r.at[j1]], buf_b, sem_b)
        a.wait()
        a2 = pltpu.async_copy(xc_hbm.at[idx_c.at[j0]], buf_a, sem_a, add=True)
        b.wait()
        b2 = pltpu.async_copy(xc_hbm.at[idx_c.at[j1]], buf_b, sem_b, add=True)
        a2.wait()
        a3 = pltpu.async_copy(buf_a, base_hbm.at[pl.ds(ebase + j0 * CH, CH)],
                              sem_a)
        b2.wait()
        b3 = pltpu.async_copy(buf_b, base_hbm.at[pl.ds(ebase + j1 * CH, CH)],
                              sem_b)
        a3.wait()
        b3.wait()
        return carry

    lax.fori_loop(0, NPAIR, pair, 0)
    t = pltpu.async_copy(xrb_hbm.at[idx_r.at[TAIL]], buf_a, sem_a)
    t.wait()
    t2 = pltpu.async_copy(xc_hbm.at[idx_c.at[TAIL]], buf_a, sem_a, add=True)
    t2.wait()
    t3 = pltpu.async_copy(buf_a, base_hbm.at[pl.ds(ebase + TAIL * CH, CH)],
                          sem_a)
    t3.wait()


def _edge_body(ea_ref, base_ref, wea_ref, w1_ref, b1_ref, out_ref):
    h = _sp(_dot(ea_ref[...], wea_ref[...]) + base_ref[...])
    out_ref[...] = _sp(_dot(h, w1_ref[...]) + b1_ref[...])


def _edge_mlp(edge_attr, base, wea, w1, b1):
    full = lambda r, c: pl.BlockSpec((r, c), lambda i: (0, 0))
    return pl.pallas_call(
        _edge_body,
        grid=(NEB,),
        in_specs=[
            pl.BlockSpec((BE, H), lambda i: (i, 0)),
            pl.BlockSpec((BE, H), lambda i: (i, 0)),
            full(H, H), full(H, H), full(1, H),
        ],
        out_specs=pl.BlockSpec((BE, H), lambda i: (i, 0)),
        out_shape=jax.ShapeDtypeStruct((E, H), jnp.float32),
    )(edge_attr, base, wea, w1, b1)


@functools.partial(
    pl.kernel,
    out_type=jax.ShapeDtypeStruct((NC, NP, H), jnp.float32),
    mesh=_MESH,
    scratch_types=[
        pltpu.VMEM_SHARED((NP, H), jnp.float32),
        pltpu.VMEM((NCH, CH), jnp.int32),
        pltpu.VMEM((CH, H), jnp.float32),
        pltpu.VMEM((CH, H), jnp.float32),
        pltpu.SemaphoreType.DMA,
        pltpu.SemaphoreType.DMA,
    ],
)
def _sc_scatter(vals_hbm, row_hbm, zrow_hbm, esum_hbm,
                acc, idx_r, buf_a, buf_b, sem_a, sem_b):
    c = lax.axis_index("c")
    s = lax.axis_index("s")
    wid = s * NC + c
    ebase = wid * EPW

    za = pltpu.async_copy(zrow_hbm, acc.at[pl.ds(s * TR, TR)], sem_a)
    pltpu.sync_copy(row_hbm.at[wid], idx_r)
    za.wait()
    plsc.subcore_barrier()

    def pair(i, carry):
        j0 = 2 * i
        j1 = j0 + 1
        a = pltpu.async_copy(vals_hbm.at[pl.ds(ebase + j0 * CH, CH)], buf_a,
                             sem_a)
        b = pltpu.async_copy(vals_hbm.at[pl.ds(ebase + j1 * CH, CH)], buf_b,
                             sem_b)
        a.wait()
        a2 = pltpu.async_copy(buf_a, acc.at[idx_r.at[j0]], sem_a, add=True)
        b.wait()
        b2 = pltpu.async_copy(buf_b, acc.at[idx_r.at[j1]], sem_b, add=True)
        a2.wait()
        b2.wait()
        return carry

    lax.fori_loop(0, NPAIR, pair, 0)
    t = pltpu.async_copy(vals_hbm.at[pl.ds(ebase + TAIL * CH, CH)], buf_a,
                         sem_a)
    t.wait()
    t2 = pltpu.async_copy(buf_a, acc.at[idx_r.at[TAIL]], sem_a, add=True)
    t2.wait()
    plsc.subcore_barrier()

    oa = pltpu.async_copy(acc.at[pl.ds(s * TR, TR)],
                          esum_hbm.at[c, pl.ds(s * TR, TR)], sem_a)
    oa.wait()


@functools.partial(
    pl.kernel,
    out_type=jax.ShapeDtypeStruct((NC, NP, CW), jnp.float32),
    mesh=_MESH,
    scratch_types=[
        pltpu.VMEM_SHARED((NP, CW), jnp.float32),
        pltpu.VMEM((NCH, CH), jnp.int32),
        pltpu.VMEM((CH, CW), jnp.float32),
        pltpu.SemaphoreType.DMA,
        pltpu.SemaphoreType.DMA,
    ],
)
def _sc_count(row_hbm, zcnt_hbm, cnt_hbm, acc_cnt, idx_r, ones_v,
              sem_a, sem_b):
    c = lax.axis_index("c")
    s = lax.axis_index("s")
    wid = s * NC + c

    za = pltpu.async_copy(zcnt_hbm, acc_cnt.at[pl.ds(s * TR, TR)], sem_a)
    pltpu.sync_copy(row_hbm.at[wid], idx_r)

    def fill(i, carry):
        ones_v[i, pl.ds(0, CW)] = jnp.ones((CW,), jnp.float32)
        return carry

    lax.fori_loop(0, CH, fill, 0)
    za.wait()
    plsc.subcore_barrier()

    def pair(i, carry):
        j0 = 2 * i
        j1 = j0 + 1
        a = pltpu.async_copy(ones_v, acc_cnt.at[idx_r.at[j0]], sem_a,
                             add=True)
        b = pltpu.async_copy(ones_v, acc_cnt.at[idx_r.at[j1]], sem_b,
                             add=True)
        a.wait()
        b.wait()
        return carry

    lax.fori_loop(0, NPAIR, pair, 0)
    t = pltpu.async_copy(ones_v, acc_cnt.at[idx_r.at[TAIL]], sem_a, add=True)
    t.wait()
    plsc.subcore_barrier()

    oa = pltpu.async_copy(acc_cnt.at[pl.ds(s * TR, TR)],
                          cnt_hbm.at[c, pl.ds(s * TR, TR)], sem_a)
    oa.wait()


def _node_body(x_ref, es_ref, cn_ref, u_ref, b_ref,
               wvx_ref, wve_ref, wvu_ref, bv0_ref, wv1_ref, bv1_ref,
               wuu_ref, wue_ref, wuv_ref, bu0_ref, wu1_ref, bu1_ref,
               xnew_ref, unew_ref,
               aes_ref, aec_ref, avs_ref, avc_ref):
    i = pl.program_id(0)

    @pl.when(i == 0)
    def _():
        aes_ref[...] = jnp.zeros((G, H), jnp.float32)
        aec_ref[...] = jnp.zeros((G, CW), jnp.float32)
        avs_ref[...] = jnp.zeros((G, H), jnp.float32)
        avc_ref[...] = jnp.zeros((G, CW), jnp.float32)

    esum = es_ref[0] + es_ref[1]
    cnt16 = cn_ref[0] + cn_ref[1]
    cnt1 = cnt16[:, 0:1]
    e_aggr = esum / jnp.maximum(cnt1, 1.0)
    b = b_ref[0]
    ohT = (lax.broadcasted_iota(jnp.int32, (G, BN), 0) == b).astype(jnp.float32)
    uv = _dot(u_ref[...], wvu_ref[...])
    h0 = (_dot(x_ref[...], wvx_ref[...]) + _dot(e_aggr, wve_ref[...])
          + _dot0(ohT, uv) + bv0_ref[...])
    xnew = _sp(_dot(_sp(h0), wv1_ref[...]) + bv1_ref[...])
    xnew_ref[...] = xnew

    aes_ref[...] += _dot(ohT, esum)
    aec_ref[...] += _dot(ohT, cnt16)
    avs_ref[...] += _dot(ohT, xnew)
    avc_ref[...] += _dot(ohT, jnp.ones((BN, CW), jnp.float32))

    @pl.when(i == NB - 1)
    def _():
        e_mean = aes_ref[...] / jnp.maximum(aec_ref[:, 0:1], 1.0)
        v_mean = avs_ref[...] / jnp.maximum(avc_ref[:, 0:1], 1.0)
        h0u = (_dot(u_ref[...], wuu_ref[...]) + _dot(e_mean, wue_ref[...])
               + _dot(v_mean, wuv_ref[...]) + bu0_ref[...])
        unew_ref[...] = _sp(_dot(_sp(h0u), wu1_ref[...]) + bu1_ref[...])


def _node(x, esum, cnt, u, batch3, wvx, wve, wvu, bv0, wv1, bv1,
          wuu, wue, wuv, bu0, wu1, bu1):
    full = lambda r, c: pl.BlockSpec((r, c), lambda i: (0, 0))
    return pl.pallas_call(
        _node_body,
        grid=(NB,),
        in_specs=[
            pl.BlockSpec((BN, H), lambda i: (i, 0)),
            pl.BlockSpec((NC, BN, H), lambda i: (0, i, 0)),
            pl.BlockSpec((NC, BN, CW), lambda i: (0, i, 0)),
            full(G, H),
            pl.BlockSpec((1, 1, BN), lambda i: (i, 0, 0)),
            full(H, H), full(H, H), full(H, H), full(1, H), full(H, H),
            full(1, H),
            full(H, H), full(H, H), full(H, H), full(1, H), full(H, H),
            full(1, H),
        ],
        out_specs=[pl.BlockSpec((BN, H), lambda i: (i, 0)),
                   pl.BlockSpec((G, H), lambda i: (0, 0))],
        out_shape=[jax.ShapeDtypeStruct((N, H), jnp.float32),
                   jax.ShapeDtypeStruct((G, H), jnp.float32)],
        scratch_shapes=[
            pltpu.VMEM((G, H), jnp.float32),
            pltpu.VMEM((G, CW), jnp.float32),
            pltpu.VMEM((G, H), jnp.float32),
            pltpu.VMEM((G, CW), jnp.float32),
        ],
    )(x, esum, cnt, u, batch3, wvx, wve, wvu, bv0, wv1, bv1,
      wuu, wue, wuv, bu0, wu1, bu1)


def kernel(x, edge_index, edge_attr, u, batch,
           phi_e_W0, phi_e_b0, phi_e_W1, phi_e_b1,
           phi_v_W0, phi_v_b0, phi_v_W1, phi_v_b1,
           phi_u_W0, phi_u_b0, phi_u_W1, phi_u_b1):
    row = edge_index[0].astype(jnp.int32)
    col = edge_index[1].astype(jnp.int32)
    batch3 = batch.astype(jnp.int32).reshape(NB, 1, BN)
    row3 = row.reshape(NW, NCH, CH)
    col3 = col.reshape(NW, NCH, CH)

    wr, wc, wea, wu = (phi_e_W0[0:H], phi_e_W0[H:2 * H],
                       phi_e_W0[2 * H:3 * H], phi_e_W0[3 * H:4 * H])
    wvx, wve, wvu = phi_v_W0[0:H], phi_v_W0[H:2 * H], phi_v_W0[2 * H:3 * H]
    wuu, wue, wuv = phi_u_W0[0:H], phi_u_W0[H:2 * H], phi_u_W0[2 * H:3 * H]
    be0 = phi_e_b0.reshape(1, H)
    be1 = phi_e_b1.reshape(1, H)
    bv0 = phi_v_b0.reshape(1, H)
    bv1 = phi_v_b1.reshape(1, H)
    bu0 = phi_u_b0.reshape(1, H)
    bu1 = phi_u_b1.reshape(1, H)

    xrb, xc = _prep(x, batch3, u, wr, wc, wu, be0)
    base = _sc_gather(xrb, xc, row3, col3)
    zcnt = jnp.zeros((TR, CW), jnp.float32)
    cnt = _sc_count(row3, zcnt)
    enew = _edge_mlp(edge_attr, base, wea, phi_e_W1, be1)
    zrow = jnp.zeros((TR, H), jnp.float32)
    esum = _sc_scatter(enew, row3, zrow)
    xnew, unew = _node(x, esum, cnt, u, batch3, wvx, wve, wvu, bv0,
                       phi_v_W1, bv1, wuu, wue, wuv, bu0, phi_u_W1, bu1)
    return xnew, enew, unew


# speedup vs baseline: 9.1536x; 1.5526x over previous
import functools

import jax
import jax.numpy as jnp
from jax import lax
from jax.experimental import pallas as pl
from jax.experimental.pallas import tpu as pltpu
from jax.experimental.pallas import tpu_sc as plsc

N = 10000
E = 320000
H = 128
G = 64

NC = 2
NS = 16
NW = NC * NS
EPW = E // NW
CH = 80
NCH = EPW // CH
NPAIR = (NCH - 1) // 2
TAIL = 2 * NPAIR
NP = 10240
TR = NP // NS
CW = 128

BN = 1000
NB = N // BN
BE = 2000
NEB = E // BE

_PREC = lax.Precision.DEFAULT


def _sp(x):
    return jnp.maximum(x, 0.0) + jnp.log1p(jnp.exp(-jnp.abs(x)))


def _dot(a, b):
    return jnp.dot(a, b, preferred_element_type=jnp.float32, precision=_PREC)


def _dot0(a, b):
    return lax.dot_general(a, b, (((0,), (0,)), ((), ())),
                           preferred_element_type=jnp.float32, precision=_PREC)


def _prep_body(x_ref, b_ref, u_ref, wr_ref, wc_ref, wu_ref, be0_ref,
               xrb_ref, xc_ref):
    ue = _dot(u_ref[...], wu_ref[...])
    b = b_ref[0]
    ohT = (lax.broadcasted_iota(jnp.int32, (G, BN), 0) == b).astype(jnp.float32)
    ubn = _dot0(ohT, ue)
    xb = x_ref[...]
    xrb_ref[...] = _dot(xb, wr_ref[...]) + ubn + be0_ref[...]
    xc_ref[...] = _dot(xb, wc_ref[...])


def _prep(x, batch3, u, wr, wc, wu, be0):
    full = lambda r, c: pl.BlockSpec((r, c), lambda i: (0, 0))
    return pl.pallas_call(
        _prep_body,
        grid=(NB,),
        in_specs=[
            pl.BlockSpec((BN, H), lambda i: (i, 0)),
            pl.BlockSpec((1, 1, BN), lambda i: (i, 0, 0)),
            full(G, H), full(H, H), full(H, H), full(H, H), full(1, H),
        ],
        out_specs=[pl.BlockSpec((BN, H), lambda i: (i, 0)),
                   pl.BlockSpec((BN, H), lambda i: (i, 0))],
        out_shape=[jax.ShapeDtypeStruct((N, H), jnp.float32)] * 2,
    )(x, batch3, u, wr, wc, wu, be0)


_MESH = plsc.VectorSubcoreMesh(core_axis_name="c", subcore_axis_name="s",
                               num_cores=NC, num_subcores=NS)


@functools.partial(
    pl.kernel,
    out_type=jax.ShapeDtypeStruct((E, H), jnp.float32),
    mesh=_MESH,
    scratch_types=[
        pltpu.VMEM((NCH, CH), jnp.int32),
        pltpu.VMEM((NCH, CH), jnp.int32),
        pltpu.VMEM((CH, H), jnp.float32),
        pltpu.VMEM((CH, H), jnp.float32),
        pltpu.SemaphoreType.DMA,
        pltpu.SemaphoreType.DMA,
    ],
)
def _sc_gather(xrb_hbm, xc_hbm, row_hbm, col_hbm, base_hbm,
               idx_r, idx_c, buf_a, buf_b, sem_a, sem_b):
    c = lax.axis_index("c")
    s = lax.axis_index("s")
    wid = s * NC + c
    ebase = wid * EPW
    pltpu.sync_copy(row_hbm.at[wid], idx_r)
    pltpu.sync_copy(col_hbm.at[wid], idx_c)

    def pair(i, carry):
        j0 = 2 * i
        j1 = j0 + 1
        a = pltpu.async_copy(xrb_hbm.at[idx_r.at[j0]], buf_a, sem_a)
        b = pltpu.async_copy(xrb_hbm.at[idx_r.at[j1]], buf_b, sem_b)
        a.wait()
        a2 = pltpu.async_copy(xc_hbm.at[idx_c.at[j0]], buf_a, sem_a, add=True)
        b.wait()
        b2 = pltpu.async_copy(xc_hbm.at[idx_c.at[j1]], buf_b, sem_b, add=True)
        a2.wait()
        a3 = pltpu.async_copy(buf_a, base_hbm.at[pl.ds(ebase + j0 * CH, CH)],
                              sem_a)
        b2.wait()
        b3 = pltpu.async_copy(buf_b, base_hbm.at[pl.ds(ebase + j1 * CH, CH)],
                              sem_b)
        a3.wait()
        b3.wait()
        return carry

    lax.fori_loop(0, NPAIR, pair, 0)
    t = pltpu.async_copy(xrb_hbm.at[idx_r.at[TAIL]], buf_a, sem_a)
    t.wait()
    t2 = pltpu.async_copy(xc_hbm.at[idx_c.at[TAIL]], buf_a, sem_a, add=True)
    t2.wait()
    t3 = pltpu.async_copy(buf_a, base_hbm.at[pl.ds(ebase + TAIL * CH, CH)],
                          sem_a)
    t3.wait()


def _edge_body(ea_ref, base_ref, wea_ref, w1_ref, b1_ref, out_ref):
    h = _sp(_dot(ea_ref[...], wea_ref[...]) + base_ref[...])
    out_ref[...] = _sp(_dot(h, w1_ref[...]) + b1_ref[...])


def _edge_mlp(edge_attr, base, wea, w1, b1):
    full = lambda r, c: pl.BlockSpec((r, c), lambda i: (0, 0))
    return pl.pallas_call(
        _edge_body,
        grid=(NEB,),
        in_specs=[
            pl.BlockSpec((BE, H), lambda i: (i, 0)),
            pl.BlockSpec((BE, H), lambda i: (i, 0)),
            full(H, H), full(H, H), full(1, H),
        ],
        out_specs=pl.BlockSpec((BE, H), lambda i: (i, 0)),
        out_shape=jax.ShapeDtypeStruct((E, H), jnp.float32),
    )(edge_attr, base, wea, w1, b1)


@functools.partial(
    pl.kernel,
    out_type=jax.ShapeDtypeStruct((NC, NP, H), jnp.float32),
    mesh=_MESH,
    scratch_types=[
        pltpu.VMEM_SHARED((NP, H), jnp.float32),
        pltpu.VMEM((NCH, CH), jnp.int32),
        pltpu.VMEM((CH, H), jnp.float32),
        pltpu.VMEM((CH, H), jnp.float32),
        pltpu.SemaphoreType.DMA,
        pltpu.SemaphoreType.DMA,
    ],
)
def _sc_scatter(vals_hbm, row_hbm, zrow_hbm, esum_hbm,
                acc, idx_r, buf_a, buf_b, sem_a, sem_b):
    c = lax.axis_index("c")
    s = lax.axis_index("s")
    wid = s * NC + c
    ebase = wid * EPW

    za = pltpu.async_copy(zrow_hbm, acc.at[pl.ds(s * TR, TR)], sem_a)
    pltpu.sync_copy(row_hbm.at[wid], idx_r)
    za.wait()
    plsc.subcore_barrier()

    def pair(i, carry):
        j0 = 2 * i
        j1 = j0 + 1
        a = pltpu.async_copy(vals_hbm.at[pl.ds(ebase + j0 * CH, CH)], buf_a,
                             sem_a)
        b = pltpu.async_copy(vals_hbm.at[pl.ds(ebase + j1 * CH, CH)], buf_b,
                             sem_b)
        a.wait()
        a2 = pltpu.async_copy(buf_a, acc.at[idx_r.at[j0]], sem_a, add=True)
        b.wait()
        b2 = pltpu.async_copy(buf_b, acc.at[idx_r.at[j1]], sem_b, add=True)
        a2.wait()
        b2.wait()
        return carry

    lax.fori_loop(0, NPAIR, pair, 0)
    t = pltpu.async_copy(vals_hbm.at[pl.ds(ebase + TAIL * CH, CH)], buf_a,
                         sem_a)
    t.wait()
    t2 = pltpu.async_copy(buf_a, acc.at[idx_r.at[TAIL]], sem_a, add=True)
    t2.wait()
    plsc.subcore_barrier()

    oa = pltpu.async_copy(acc.at[pl.ds(s * TR, TR)],
                          esum_hbm.at[c, pl.ds(s * TR, TR)], sem_a)
    oa.wait()


@functools.partial(
    pl.kernel,
    out_type=jax.ShapeDtypeStruct((NC, NP, CW), jnp.float32),
    mesh=_MESH,
    scratch_types=[
        pltpu.VMEM_SHARED((NP, CW), jnp.float32),
        pltpu.VMEM((NCH, CH), jnp.int32),
        pltpu.VMEM((CH, CW), jnp.float32),
        pltpu.SemaphoreType.DMA,
        pltpu.SemaphoreType.DMA,
    ],
)
def _sc_count(row_hbm, zcnt_hbm, cnt_hbm, acc_cnt, idx_r, ones_v,
              sem_a, sem_b):
    c = lax.axis_index("c")
    s = lax.axis_index("s")
    wid = s * NC + c

    za = pltpu.async_copy(zcnt_hbm, acc_cnt.at[pl.ds(s * TR, TR)], sem_a)
    pltpu.sync_copy(row_hbm.at[wid], idx_r)

    def fill(i, carry):
        ones_v[i, pl.ds(0, CW)] = jnp.ones((CW,), jnp.float32)
        return carry

    lax.fori_loop(0, CH, fill, 0)
    za.wait()
    plsc.subcore_barrier()

    def pair(i, carry):
        j0 = 2 * i
        j1 = j0 + 1
        a = pltpu.async_copy(ones_v, acc_cnt.at[idx_r.at[j0]], sem_a,
                             add=True)
        b = pltpu.async_copy(ones_v, acc_cnt.at[idx_r.at[j1]], sem_b,
                             add=True)
        a.wait()
        b.wait()
        return carry

    lax.fori_loop(0, NPAIR, pair, 0)
    t = pltpu.async_copy(ones_v, acc_cnt.at[idx_r.at[TAIL]], sem_a, add=True)
    t.wait()
    plsc.subcore_barrier()

    oa = pltpu.async_copy(acc_cnt.at[pl.ds(s * TR, TR)],
                          cnt_hbm.at[c, pl.ds(s * TR, TR)], sem_a)
    oa.wait()


def _node_body(x_ref, es_ref, cn_ref, u_ref, b_ref,
               wvx_ref, wve_ref, wvu_ref, bv0_ref, wv1_ref, bv1_ref,
               wuu_ref, wue_ref, wuv_ref, bu0_ref, wu1_ref, bu1_ref,
               xnew_ref, unew_ref,
               aes_ref, aec_ref, avs_ref, avc_ref):
    i = pl.program_id(0)

    @pl.when(i == 0)
    def _():
        aes_ref[...] = jnp.zeros((G, H), jnp.float32)
        aec_ref[...] = jnp.zeros((G, CW), jnp.float32)
        avs_ref[...] = jnp.zeros((G, H), jnp.float32)
        avc_ref[...] = jnp.zeros((G, CW), jnp.float32)

    esum = es_ref[0] + es_ref[1]
    cnt16 = cn_ref[0] + cn_ref[1]
    cnt1 = cnt16[:, 0:1]
    e_aggr = esum / jnp.maximum(cnt1, 1.0)
    b = b_ref[0]
    ohT = (lax.broadcasted_iota(jnp.int32, (G, BN), 0) == b).astype(jnp.float32)
    uv = _dot(u_ref[...], wvu_ref[...])
    h0 = (_dot(x_ref[...], wvx_ref[...]) + _dot(e_aggr, wve_ref[...])
          + _dot0(ohT, uv) + bv0_ref[...])
    xnew = _sp(_dot(_sp(h0), wv1_ref[...]) + bv1_ref[...])
    xnew_ref[...] = xnew

    aes_ref[...] += _dot(ohT, esum)
    aec_ref[...] += _dot(ohT, cnt16)
    avs_ref[...] += _dot(ohT, xnew)
    avc_ref[...] += _dot(ohT, jnp.ones((BN, CW), jnp.float32))

    @pl.when(i == NB - 1)
    def _():
        e_mean = aes_ref[...] / jnp.maximum(aec_ref[:, 0:1], 1.0)
        v_mean = avs_ref[...] / jnp.maximum(avc_ref[:, 0:1], 1.0)
        h0u = (_dot(u_ref[...], wuu_ref[...]) + _dot(e_mean, wue_ref[...])
               + _dot(v_mean, wuv_ref[...]) + bu0_ref[...])
        unew_ref[...] = _sp(_dot(_sp(h0u), wu1_ref[...]) + bu1_ref[...])


def _node(x, esum, cnt, u, batch3, wvx, wve, wvu, bv0, wv1, bv1,
          wuu, wue, wuv, bu0, wu1, bu1):
    full = lambda r, c: pl.BlockSpec((r, c), lambda i: (0, 0))
    return pl.pallas_call(
        _node_body,
        grid=(NB,),
        in_specs=[
            pl.BlockSpec((BN, H), lambda i: (i, 0)),
            pl.BlockSpec((NC, BN, H), lambda i: (0, i, 0)),
            pl.BlockSpec((NC, BN, CW), lambda i: (0, i, 0)),
            full(G, H),
            pl.BlockSpec((1, 1, BN), lambda i: (i, 0, 0)),
            full(H, H), full(H, H), full(H, H), full(1, H), full(H, H),
            full(1, H),
            full(H, H), full(H, H), full(H, H), full(1, H), full(H, H),
            full(1, H),
        ],
        out_specs=[pl.BlockSpec((BN, H), lambda i: (i, 0)),
                   pl.BlockSpec((G, H), lambda i: (0, 0))],
        out_shape=[jax.ShapeDtypeStruct((N, H), jnp.float32),
                   jax.ShapeDtypeStruct((G, H), jnp.float32)],
        scratch_shapes=[
            pltpu.VMEM((G, H), jnp.float32),
            pltpu.VMEM((G, CW), jnp.float32),
            pltpu.VMEM((G, H), jnp.float32),
            pltpu.VMEM((G, CW), jnp.float32),
        ],
    )(x, esum, cnt, u, batch3, wvx, wve, wvu, bv0, wv1, bv1,
      wuu, wue, wuv, bu0, wu1, bu1)


def kernel(x, edge_index, edge_attr, u, batch,
           phi_e_W0, phi_e_b0, phi_e_W1, phi_e_b1,
           phi_v_W0, phi_v_b0, phi_v_W1, phi_v_b1,
           phi_u_W0, phi_u_b0, phi_u_W1, phi_u_b1):
    row = edge_index[0].astype(jnp.int32)
    col = edge_index[1].astype(jnp.int32)
    batch3 = batch.astype(jnp.int32).reshape(NB, 1, BN)
    row3 = row.reshape(NW, NCH, CH)
    col3 = col.reshape(NW, NCH, CH)

    wr, wc, wea, wu = (phi_e_W0[0:H], phi_e_W0[H:2 * H],
                       phi_e_W0[2 * H:3 * H], phi_e_W0[3 * H:4 * H])
    wvx, wve, wvu = phi_v_W0[0:H], phi_v_W0[H:2 * H], phi_v_W0[2 * H:3 * H]
    wuu, wue, wuv = phi_u_W0[0:H], phi_u_W0[H:2 * H], phi_u_W0[2 * H:3 * H]
    be0 = phi_e_b0.reshape(1, H)
    be1 = phi_e_b1.reshape(1, H)
    bv0 = phi_v_b0.reshape(1, H)
    bv1 = phi_v_b1.reshape(1, H)
    bu0 = phi_u_b0.reshape(1, H)
    bu1 = phi_u_b1.reshape(1, H)

    xrb, xc = _prep(x, batch3, u, wr, wc, wu, be0)
    base = _sc_gather(xrb, xc, row3, col3)
    zcnt = jnp.zeros((TR, CW), jnp.float32)
    cnt = _sc_count(row3, zcnt)
    enew = _edge_mlp(edge_attr, base, wea, phi_e_W1, be1)
    zrow = jnp.zeros((TR, H), jnp.float32)
    esum = _sc_scatter(enew, row3, zrow)
    xnew, unew = _node(x, esum, cnt, u, batch3, wvx, wve, wvu, bv0,
                       phi_v_W1, bv1, wuu, wue, wuv, bu0, phi_u_W1, bu1)
    return xnew, enew, unew


# speedup vs baseline: 9.6662x; 1.0560x over previous
import functools

import jax
import jax.numpy as jnp
from jax import lax
from jax.experimental import pallas as pl
from jax.experimental.pallas import tpu as pltpu
from jax.experimental.pallas import tpu_sc as plsc

N = 10000
E = 320000
H = 128
G = 64

NC = 2
NS = 16
NW = NC * NS
EPW = E // NW
CH = 80
NCH = EPW // CH
NPAIR = (NCH - 1) // 2
TAIL = 2 * NPAIR

S = 5
ES = E // S
EPWS = ES // NW
NCHS = EPWS // CH
PAIRS_S = (NCHS - 1) // 2
TAIL_S = 2 * PAIRS_S
NP = 10240
TR = NP // NS
CW = 128

BN = 1000
NB = N // BN
BE = 2000
NEB = E // BE

_PREC = lax.Precision.DEFAULT


def _sp(x):
    return jnp.maximum(x, 0.0) + jnp.log1p(jnp.exp(-jnp.abs(x)))


def _dot(a, b):
    return jnp.dot(a, b, preferred_element_type=jnp.float32, precision=_PREC)


def _dot0(a, b):
    return lax.dot_general(a, b, (((0,), (0,)), ((), ())),
                           preferred_element_type=jnp.float32, precision=_PREC)


def _prep_body(x_ref, b_ref, u_ref, wr_ref, wc_ref, wu_ref, be0_ref,
               xrb_ref, xc_ref):
    ue = _dot(u_ref[...], wu_ref[...])
    b = b_ref[0]
    ohT = (lax.broadcasted_iota(jnp.int32, (G, BN), 0) == b).astype(jnp.float32)
    ubn = _dot0(ohT, ue)
    xb = x_ref[...]
    xrb_ref[...] = _dot(xb, wr_ref[...]) + ubn + be0_ref[...]
    xc_ref[...] = _dot(xb, wc_ref[...])


def _prep(x, batch3, u, wr, wc, wu, be0):
    full = lambda r, c: pl.BlockSpec((r, c), lambda i: (0, 0))
    return pl.pallas_call(
        _prep_body,
        grid=(NB,),
        in_specs=[
            pl.BlockSpec((BN, H), lambda i: (i, 0)),
            pl.BlockSpec((1, 1, BN), lambda i: (i, 0, 0)),
            full(G, H), full(H, H), full(H, H), full(H, H), full(1, H),
        ],
        out_specs=[pl.BlockSpec((BN, H), lambda i: (i, 0)),
                   pl.BlockSpec((BN, H), lambda i: (i, 0))],
        out_shape=[jax.ShapeDtypeStruct((N, H), jnp.float32)] * 2,
    )(x, batch3, u, wr, wc, wu, be0)


_MESH = plsc.VectorSubcoreMesh(core_axis_name="c", subcore_axis_name="s",
                               num_cores=NC, num_subcores=NS)


@functools.partial(
    pl.kernel,
    out_type=jax.ShapeDtypeStruct((ES, H), jnp.float32),
    mesh=_MESH,
    scratch_types=[
        pltpu.VMEM((NCHS, CH), jnp.int32),
        pltpu.VMEM((NCHS, CH), jnp.int32),
        pltpu.VMEM((CH, H), jnp.float32),
        pltpu.VMEM((CH, H), jnp.float32),
        pltpu.SemaphoreType.DMA,
        pltpu.SemaphoreType.DMA,
    ],
)
def _sc_gather(xrb_hbm, xc_hbm, row_hbm, col_hbm, base_hbm,
               idx_r, idx_c, buf_a, buf_b, sem_a, sem_b):
    c = lax.axis_index("c")
    s = lax.axis_index("s")
    wid = s * NC + c
    ebase = wid * EPWS
    pltpu.sync_copy(row_hbm.at[wid], idx_r)
    pltpu.sync_copy(col_hbm.at[wid], idx_c)

    def pair(i, carry):
        j0 = 2 * i
        j1 = j0 + 1
        a = pltpu.async_copy(xrb_hbm.at[idx_r.at[j0]], buf_a, sem_a)
        b = pltpu.async_copy(xrb_hbm.at[idx_r.at[j1]], buf_b, sem_b)
        a.wait()
        a2 = pltpu.async_copy(xc_hbm.at[idx_c.at[j0]], buf_a, sem_a, add=True)
        b.wait()
        b2 = pltpu.async_copy(xc_hbm.at[idx_c.at[j1]], buf_b, sem_b, add=True)
        a2.wait()
        a3 = pltpu.async_copy(buf_a, base_hbm.at[pl.ds(ebase + j0 * CH, CH)],
                              sem_a)
        b2.wait()
        b3 = pltpu.async_copy(buf_b, base_hbm.at[pl.ds(ebase + j1 * CH, CH)],
                              sem_b)
        a3.wait()
        b3.wait()
        return carry

    lax.fori_loop(0, PAIRS_S, pair, 0)
    t = pltpu.async_copy(xrb_hbm.at[idx_r.at[TAIL_S]], buf_a, sem_a)
    t.wait()
    t2 = pltpu.async_copy(xc_hbm.at[idx_c.at[TAIL_S]], buf_a, sem_a, add=True)
    t2.wait()
    t3 = pltpu.async_copy(buf_a, base_hbm.at[pl.ds(ebase + TAIL_S * CH, CH)],
                          sem_a)
    t3.wait()


def _edge_body(ea_ref, base_ref, wea_ref, w1_ref, b1_ref, out_ref):
    h = _sp(_dot(ea_ref[...], wea_ref[...]) + base_ref[...])
    out_ref[...] = _sp(_dot(h, w1_ref[...]) + b1_ref[...])


NEBS = ES // BE


def _edge_mlp_seg(seg, edge_attr, base, wea, w1, b1):
    off = seg * NEBS
    full = lambda r, c: pl.BlockSpec((r, c), lambda i: (0, 0))
    return pl.pallas_call(
        _edge_body,
        grid=(NEBS,),
        in_specs=[
            pl.BlockSpec((BE, H), lambda i, _o=off: (_o + i, 0)),
            pl.BlockSpec((BE, H), lambda i: (i, 0)),
            full(H, H), full(H, H), full(1, H),
        ],
        out_specs=pl.BlockSpec((BE, H), lambda i: (i, 0)),
        out_shape=jax.ShapeDtypeStruct((ES, H), jnp.float32),
    )(edge_attr, base, wea, w1, b1)


@functools.partial(
    pl.kernel,
    out_type=jax.ShapeDtypeStruct((NC, NP, H), jnp.float32),
    mesh=_MESH,
    scratch_types=[
        pltpu.VMEM_SHARED((NP, H), jnp.float32),
        pltpu.VMEM((NCHS, CH), jnp.int32),
        pltpu.VMEM((CH, H), jnp.float32),
        pltpu.VMEM((CH, H), jnp.float32),
        pltpu.SemaphoreType.DMA,
        pltpu.SemaphoreType.DMA,
    ],
)
def _sc_scatter(v0_hbm, v1_hbm, v2_hbm, v3_hbm, v4_hbm, row_hbm, zrow_hbm,
                esum_hbm, acc, idx_r, buf_a, buf_b, sem_a, sem_b):
    c = lax.axis_index("c")
    s = lax.axis_index("s")
    wid = s * NC + c
    ebase = wid * EPWS

    za = pltpu.async_copy(zrow_hbm, acc.at[pl.ds(s * TR, TR)], sem_a)
    za.wait()
    plsc.subcore_barrier()

    for seg, vals_hbm in enumerate((v0_hbm, v1_hbm, v2_hbm, v3_hbm, v4_hbm)):
        pltpu.sync_copy(row_hbm.at[seg, wid], idx_r)

        def pair(i, carry, vals_hbm=vals_hbm):
            j0 = 2 * i
            j1 = j0 + 1
            a = pltpu.async_copy(vals_hbm.at[pl.ds(ebase + j0 * CH, CH)],
                                 buf_a, sem_a)
            b = pltpu.async_copy(vals_hbm.at[pl.ds(ebase + j1 * CH, CH)],
                                 buf_b, sem_b)
            a.wait()
            a2 = pltpu.async_copy(buf_a, acc.at[idx_r.at[j0]], sem_a,
                                  add=True)
            b.wait()
            b2 = pltpu.async_copy(buf_b, acc.at[idx_r.at[j1]], sem_b,
                                  add=True)
            a2.wait()
            b2.wait()
            return carry

        lax.fori_loop(0, PAIRS_S, pair, 0)
        t = pltpu.async_copy(vals_hbm.at[pl.ds(ebase + TAIL_S * CH, CH)],
                             buf_a, sem_a)
        t.wait()
        t2 = pltpu.async_copy(buf_a, acc.at[idx_r.at[TAIL_S]], sem_a,
                              add=True)
        t2.wait()

    plsc.subcore_barrier()
    oa = pltpu.async_copy(acc.at[pl.ds(s * TR, TR)],
                          esum_hbm.at[c, pl.ds(s * TR, TR)], sem_a)
    oa.wait()


@functools.partial(
    pl.kernel,
    out_type=jax.ShapeDtypeStruct((NC, NP, CW), jnp.float32),
    mesh=_MESH,
    scratch_types=[
        pltpu.VMEM_SHARED((NP, CW), jnp.float32),
        pltpu.VMEM((NCH, CH), jnp.int32),
        pltpu.VMEM((CH, CW), jnp.float32),
        pltpu.SemaphoreType.DMA,
        pltpu.SemaphoreType.DMA,
    ],
)
def _sc_count(row_hbm, zcnt_hbm, cnt_hbm, acc_cnt, idx_r, ones_v,
              sem_a, sem_b):
    c = lax.axis_index("c")
    s = lax.axis_index("s")
    wid = s * NC + c

    za = pltpu.async_copy(zcnt_hbm, acc_cnt.at[pl.ds(s * TR, TR)], sem_a)
    pltpu.sync_copy(row_hbm.at[wid], idx_r)

    def fill(i, carry):
        ones_v[i, pl.ds(0, CW)] = jnp.ones((CW,), jnp.float32)
        return carry

    lax.fori_loop(0, CH, fill, 0)
    za.wait()
    plsc.subcore_barrier()

    def pair(i, carry):
        j0 = 2 * i
        j1 = j0 + 1
        a = pltpu.async_copy(ones_v, acc_cnt.at[idx_r.at[j0]], sem_a,
                             add=True)
        b = pltpu.async_copy(ones_v, acc_cnt.at[idx_r.at[j1]], sem_b,
                             add=True)
        a.wait()
        b.wait()
        return carry

    lax.fori_loop(0, NPAIR, pair, 0)
    t = pltpu.async_copy(ones_v, acc_cnt.at[idx_r.at[TAIL]], sem_a, add=True)
    t.wait()
    plsc.subcore_barrier()

    oa = pltpu.async_copy(acc_cnt.at[pl.ds(s * TR, TR)],
                          cnt_hbm.at[c, pl.ds(s * TR, TR)], sem_a)
    oa.wait()


def _node_body(x_ref, es_ref, cn_ref, u_ref, b_ref,
               wvx_ref, wve_ref, wvu_ref, bv0_ref, wv1_ref, bv1_ref,
               wuu_ref, wue_ref, wuv_ref, bu0_ref, wu1_ref, bu1_ref,
               xnew_ref, unew_ref,
               aes_ref, aec_ref, avs_ref, avc_ref):
    i = pl.program_id(0)

    @pl.when(i == 0)
    def _():
        aes_ref[...] = jnp.zeros((G, H), jnp.float32)
        aec_ref[...] = jnp.zeros((G, CW), jnp.float32)
        avs_ref[...] = jnp.zeros((G, H), jnp.float32)
        avc_ref[...] = jnp.zeros((G, CW), jnp.float32)

    esum = es_ref[0] + es_ref[1]
    cnt16 = cn_ref[0] + cn_ref[1]
    cnt1 = cnt16[:, 0:1]
    e_aggr = esum / jnp.maximum(cnt1, 1.0)
    b = b_ref[0]
    ohT = (lax.broadcasted_iota(jnp.int32, (G, BN), 0) == b).astype(jnp.float32)
    uv = _dot(u_ref[...], wvu_ref[...])
    h0 = (_dot(x_ref[...], wvx_ref[...]) + _dot(e_aggr, wve_ref[...])
          + _dot0(ohT, uv) + bv0_ref[...])
    xnew = _sp(_dot(_sp(h0), wv1_ref[...]) + bv1_ref[...])
    xnew_ref[...] = xnew

    aes_ref[...] += _dot(ohT, esum)
    aec_ref[...] += _dot(ohT, cnt16)
    avs_ref[...] += _dot(ohT, xnew)
    avc_ref[...] += _dot(ohT, jnp.ones((BN, CW), jnp.float32))

    @pl.when(i == NB - 1)
    def _():
        e_mean = aes_ref[...] / jnp.maximum(aec_ref[:, 0:1], 1.0)
        v_mean = avs_ref[...] / jnp.maximum(avc_ref[:, 0:1], 1.0)
        h0u = (_dot(u_ref[...], wuu_ref[...]) + _dot(e_mean, wue_ref[...])
               + _dot(v_mean, wuv_ref[...]) + bu0_ref[...])
        unew_ref[...] = _sp(_dot(_sp(h0u), wu1_ref[...]) + bu1_ref[...])


def _node(x, esum, cnt, u, batch3, wvx, wve, wvu, bv0, wv1, bv1,
          wuu, wue, wuv, bu0, wu1, bu1):
    full = lambda r, c: pl.BlockSpec((r, c), lambda i: (0, 0))
    return pl.pallas_call(
        _node_body,
        grid=(NB,),
        in_specs=[
            pl.BlockSpec((BN, H), lambda i: (i, 0)),
            pl.BlockSpec((NC, BN, H), lambda i: (0, i, 0)),
            pl.BlockSpec((NC, BN, CW), lambda i: (0, i, 0)),
            full(G, H),
            pl.BlockSpec((1, 1, BN), lambda i: (i, 0, 0)),
            full(H, H), full(H, H), full(H, H), full(1, H), full(H, H),
            full(1, H),
            full(H, H), full(H, H), full(H, H), full(1, H), full(H, H),
            full(1, H),
        ],
        out_specs=[pl.BlockSpec((BN, H), lambda i: (i, 0)),
                   pl.BlockSpec((G, H), lambda i: (0, 0))],
        out_shape=[jax.ShapeDtypeStruct((N, H), jnp.float32),
                   jax.ShapeDtypeStruct((G, H), jnp.float32)],
        scratch_shapes=[
            pltpu.VMEM((G, H), jnp.float32),
            pltpu.VMEM((G, CW), jnp.float32),
            pltpu.VMEM((G, H), jnp.float32),
            pltpu.VMEM((G, CW), jnp.float32),
        ],
    )(x, esum, cnt, u, batch3, wvx, wve, wvu, bv0, wv1, bv1,
      wuu, wue, wuv, bu0, wu1, bu1)


def kernel(x, edge_index, edge_attr, u, batch,
           phi_e_W0, phi_e_b0, phi_e_W1, phi_e_b1,
           phi_v_W0, phi_v_b0, phi_v_W1, phi_v_b1,
           phi_u_W0, phi_u_b0, phi_u_W1, phi_u_b1):
    row = edge_index[0].astype(jnp.int32)
    col = edge_index[1].astype(jnp.int32)
    batch3 = batch.astype(jnp.int32).reshape(NB, 1, BN)
    row3 = row.reshape(NW, NCH, CH)
    row5 = row.reshape(S, NW, NCHS, CH)
    col5 = col.reshape(S, NW, NCHS, CH)

    wr, wc, wea, wu = (phi_e_W0[0:H], phi_e_W0[H:2 * H],
                       phi_e_W0[2 * H:3 * H], phi_e_W0[3 * H:4 * H])
    wvx, wve, wvu = phi_v_W0[0:H], phi_v_W0[H:2 * H], phi_v_W0[2 * H:3 * H]
    wuu, wue, wuv = phi_u_W0[0:H], phi_u_W0[H:2 * H], phi_u_W0[2 * H:3 * H]
    be0 = phi_e_b0.reshape(1, H)
    be1 = phi_e_b1.reshape(1, H)
    bv0 = phi_v_b0.reshape(1, H)
    bv1 = phi_v_b1.reshape(1, H)
    bu0 = phi_u_b0.reshape(1, H)
    bu1 = phi_u_b1.reshape(1, H)

    xrb, xc = _prep(x, batch3, u, wr, wc, wu, be0)
    bases = [_sc_gather(xrb, xc, row5[s], col5[s]) for s in range(S)]
    zcnt = jnp.zeros((TR, CW), jnp.float32)
    cnt = _sc_count(row3, zcnt)
    es = [_edge_mlp_seg(s, edge_attr, bases[s], wea, phi_e_W1, be1)
          for s in range(S)]
    enew = jnp.concatenate(es, axis=0)
    zrow = jnp.zeros((TR, H), jnp.float32)
    esum = _sc_scatter(es[0], es[1], es[2], es[3], es[4], row5, zrow)
    xnew, unew = _node(x, esum, cnt, u, batch3, wvx, wve, wvu, bv0,
                       phi_v_W1, bv1, wuu, wue, wuv, bu0, phi_u_W1, bu1)
    return xnew, enew, unew
